# baseline (speedup 1.0000x reference)
import functools

import jax
import jax.numpy as jnp
from jax import lax
from jax.experimental import pallas as pl
from jax.experimental.pallas import tpu as pltpu
from jax.experimental.pallas import tpu_sc as plsc

VOCAB = 100000
EMB = 128
BATCH = 4096
SEQ = 50
N = BATCH * SEQ
NC = 2
NS = 16
NW = NC * NS
PER_W = N // NW
CHUNK = 128
NCHUNK = PER_W // CHUNK
NBUF = 5
LOOK = 5


@functools.partial(
    pl.kernel,
    out_type=jax.ShapeDtypeStruct((N, EMB), jnp.float32),
    mesh=plsc.VectorSubcoreMesh(core_axis_name="c", subcore_axis_name="s"),
    scratch_types=[
        pltpu.VMEM((NCHUNK, CHUNK), jnp.int32),
        *[pltpu.VMEM((CHUNK, EMB), jnp.float32) for _ in range(NBUF)],
        *[pltpu.SemaphoreType.DMA for _ in range(2 * NBUF)],
    ],
)
def _emb_lookup(table_hbm, idx_hbm, out_hbm, idx_v, *bufs):
    rows = bufs[:NBUF]
    sg = bufs[NBUF:2 * NBUF]
    sw = bufs[2 * NBUF:]
    wid = lax.axis_index("s") * NC + lax.axis_index("c")
    base = wid * PER_W

    pltpu.sync_copy(idx_hbm.at[wid], idx_v)

    def gather(c):
        s = c % NBUF
        pltpu.async_copy(table_hbm.at[idx_v.at[c]], rows[s], sg[s])

    def wait_gather(c):
        s = c % NBUF
        pltpu.make_async_copy(table_hbm.at[idx_v.at[c]], rows[s], sg[s]).wait()

    def write(c):
        s = c % NBUF
        dst = out_hbm.at[pl.ds(base + c * CHUNK, CHUNK)]
        pltpu.async_copy(rows[s], dst, sw[s])

    def wait_write(c):
        s = c % NBUF
        dst = out_hbm.at[pl.ds(base + c * CHUNK, CHUNK)]
        pltpu.make_async_copy(rows[s], dst, sw[s]).wait()

    for c in range(LOOK):
        gather(c)

    for j in range(NCHUNK):
        wait_gather(j)
        write(j)
        k = j + LOOK
        if k < NCHUNK:
            if k >= NBUF:
                wait_write(k - NBUF)
            gather(k)

    for c in range(NCHUNK - NBUF, NCHUNK):
        wait_write(c)


def kernel(x, table):
    idx = x.T.reshape(NW, NCHUNK, CHUNK)
    out = _emb_lookup(table, idx)
    return out.reshape(SEQ, BATCH, EMB).transpose(1, 0, 2)


# speedup vs baseline: 1.0311x; 1.0311x over previous
import functools

import jax
import jax.numpy as jnp
from jax import lax
from jax.experimental import pallas as pl
from jax.experimental.pallas import tpu as pltpu
from jax.experimental.pallas import tpu_sc as plsc

VOCAB = 100000
EMB = 128
BATCH = 4096
SEQ = 50
N = BATCH * SEQ
NC = 2
NS = 16
NW = NC * NS
PER_W = N // NW
CHUNK = 128
NCHUNK = PER_W // CHUNK
NBUF = 5
NGROUP = NCHUNK // NBUF


@functools.partial(
    pl.kernel,
    out_type=jax.ShapeDtypeStruct((N, EMB), jnp.float32),
    mesh=plsc.VectorSubcoreMesh(core_axis_name="c", subcore_axis_name="s"),
    scratch_types=[
        pltpu.VMEM((NCHUNK, CHUNK), jnp.int32),
        *[pltpu.VMEM((CHUNK, EMB), jnp.float32) for _ in range(NBUF)],
        *[pltpu.SemaphoreType.DMA for _ in range(2 * NBUF)],
    ],
)
def _emb_lookup(table_hbm, idx_hbm, out_hbm, idx_v, *bufs):
    rows = bufs[:NBUF]
    sg = bufs[NBUF:2 * NBUF]
    sw = bufs[2 * NBUF:]
    wid = lax.axis_index("s") * NC + lax.axis_index("c")
    base = wid * PER_W

    pltpu.sync_copy(idx_hbm.at[wid], idx_v)

    for b in range(NBUF):
        pltpu.async_copy(table_hbm.at[idx_v.at[b]], rows[b], sg[b])

    def group(g, carry):
        for b in range(NBUF):
            j = g * NBUF + b
            pltpu.make_async_copy(
                table_hbm.at[idx_v.at[j]], rows[b], sg[b]).wait()
            dst = out_hbm.at[pl.ds(base + j * CHUNK, CHUNK)]
            pltpu.async_copy(rows[b], dst, sw[b])
            pltpu.make_async_copy(rows[b], dst, sw[b]).wait()
            pltpu.async_copy(table_hbm.at[idx_v.at[j + NBUF]], rows[b], sg[b])
        return carry

    lax.fori_loop(0, NGROUP - 1, group, 0)

    for b in range(NBUF):
        j = (NGROUP - 1) * NBUF + b
        pltpu.make_async_copy(table_hbm.at[idx_v.at[j]], rows[b], sg[b]).wait()
        dst = out_hbm.at[pl.ds(base + j * CHUNK, CHUNK)]
        pltpu.async_copy(rows[b], dst, sw[b])
    for b in range(NBUF):
        j = (NGROUP - 1) * NBUF + b
        dst = out_hbm.at[pl.ds(base + j * CHUNK, CHUNK)]
        pltpu.make_async_copy(rows[b], dst, sw[b]).wait()


def kernel(x, table):
    idx = x.T.reshape(NW, NCHUNK, CHUNK)
    out = _emb_lookup(table, idx)
    return out.reshape(SEQ, BATCH, EMB).transpose(1, 0, 2)


# speedup vs baseline: 1.0396x; 1.0082x over previous
import functools

import jax
import jax.numpy as jnp
from jax import lax
from jax.experimental import pallas as pl
from jax.experimental.pallas import tpu as pltpu
from jax.experimental.pallas import tpu_sc as plsc

VOCAB = 100000
EMB = 128
BATCH = 4096
SEQ = 50
N = BATCH * SEQ
NC = 2
NS = 16
NW = NC * NS
PER_W = N // NW
CHUNK = 64
NCHUNK = PER_W // CHUNK
NBUF = 10
NGROUP = NCHUNK // NBUF


@functools.partial(
    pl.kernel,
    out_type=jax.ShapeDtypeStruct((N, EMB), jnp.float32),
    mesh=plsc.VectorSubcoreMesh(core_axis_name="c", subcore_axis_name="s"),
    scratch_types=[
        pltpu.VMEM((NCHUNK, CHUNK), jnp.int32),
        *[pltpu.VMEM((CHUNK, EMB), jnp.float32) for _ in range(NBUF)],
        *[pltpu.SemaphoreType.DMA for _ in range(2 * NBUF)],
    ],
)
def _emb_lookup(table_hbm, idx_hbm, out_hbm, idx_v, *bufs):
    rows = bufs[:NBUF]
    sg = bufs[NBUF:2 * NBUF]
    sw = bufs[2 * NBUF:]
    wid = lax.axis_index("s") * NC + lax.axis_index("c")
    base = wid * PER_W

    pltpu.sync_copy(idx_hbm.at[wid], idx_v)

    for b in range(NBUF):
        pltpu.async_copy(table_hbm.at[idx_v.at[b]], rows[b], sg[b])

    def group(g, carry):
        for b in range(NBUF):
            j = g * NBUF + b
            pltpu.make_async_copy(
                table_hbm.at[idx_v.at[j]], rows[b], sg[b]).wait()
            dst = out_hbm.at[pl.ds(base + j * CHUNK, CHUNK)]
            pltpu.async_copy(rows[b], dst, sw[b])
            pltpu.make_async_copy(rows[b], dst, sw[b]).wait()
            pltpu.async_copy(table_hbm.at[idx_v.at[j + NBUF]], rows[b], sg[b])
        return carry

    lax.fori_loop(0, NGROUP - 1, group, 0)

    for b in range(NBUF):
        j = (NGROUP - 1) * NBUF + b
        pltpu.make_async_copy(table_hbm.at[idx_v.at[j]], rows[b], sg[b]).wait()
        dst = out_hbm.at[pl.ds(base + j * CHUNK, CHUNK)]
        pltpu.async_copy(rows[b], dst, sw[b])
    for b in range(NBUF):
        j = (NGROUP - 1) * NBUF + b
        dst = out_hbm.at[pl.ds(base + j * CHUNK, CHUNK)]
        pltpu.make_async_copy(rows[b], dst, sw[b]).wait()


def kernel(x, table):
    idx = x.T.reshape(NW, NCHUNK, CHUNK)
    out = _emb_lookup(table, idx)
    return out.reshape(SEQ, BATCH, EMB).transpose(1, 0, 2)
